# baseline (speedup 1.0000x reference)
import functools

import jax
import jax.numpy as jnp
from jax import lax
from jax.experimental import pallas as pl
from jax.experimental.pallas import tpu as pltpu
from jax.experimental.pallas import tpu_sc as plsc

HIST = 200
D = 16
CHUNKS = ((0, 128), (128, 72))


def _ehr_sc(idx, emb, w16, b16):
    mesh = plsc.VectorSubcoreMesh(core_axis_name="c", subcore_axis_name="s")

    @functools.partial(
        pl.kernel,
        mesh=mesh,
        compiler_params=pltpu.CompilerParams(use_tc_tiling_on_sc=False),
        out_type=[
            jax.ShapeDtypeStruct((D,), jnp.float32),
            jax.ShapeDtypeStruct((D,), jnp.float32),
        ],
        scratch_types=[
            pltpu.VMEM((HIST,), jnp.int32),
            pltpu.VMEM((HIST, D), jnp.float32),
            pltpu.VMEM((D,), jnp.float32),
            pltpu.VMEM((D,), jnp.float32),
            pltpu.VMEM((D,), jnp.float32),
            pltpu.VMEM((D,), jnp.float32),
            pltpu.VMEM((D,), jnp.float32),
            pltpu.SemaphoreType.DMA,
        ],
    )
    def k(idx_hbm, emb_hbm, w_hbm, b_hbm, emb_out, sig_out,
          idx_v, rows_v, w_v, b_v, acc_v, sig_v, prod_v, sem):
        cid = lax.axis_index("c")
        sid = lax.axis_index("s")

        @pl.when(jnp.logical_and(cid == 0, sid == 0))
        def _():
            pltpu.sync_copy(idx_hbm, idx_v)
            copies = [
                pltpu.async_copy(
                    emb_hbm.at[idx_v.at[pl.ds(base, n)]],
                    rows_v.at[pl.ds(base, n)],
                    sem,
                )
                for base, n in CHUNKS
            ]
            pltpu.sync_copy(w_hbm, w_v)
            pltpu.sync_copy(b_hbm, b_v)
            for c in copies:
                c.wait()
            accs = [rows_v[i] for i in range(8)]
            for i in range(8, HIST):
                accs[i % 8] = accs[i % 8] + rows_v[i]
            accs = [accs[0] + accs[4], accs[1] + accs[5],
                    accs[2] + accs[6], accs[3] + accs[7]]
            accs = [accs[0] + accs[2], accs[1] + accs[3]]
            acc = accs[0] + accs[1]
            acc_v[...] = acc
            pltpu.sync_copy(acc_v, emb_out)
            prod = acc * w_v[...]
            lanes = [prod[i] for i in range(D)]
            while len(lanes) > 1:
                lanes = [lanes[i] + lanes[i + 1] for i in range(0, len(lanes), 2)]
            s = lanes[0]
            x = jnp.full((D,), s, jnp.float32) + b_v[...]
            sig_v[...] = 1.0 / (1.0 + jnp.exp(-x))
            pltpu.sync_copy(sig_v, sig_out)

    return k(idx, emb, w16, b16)


def kernel(label, ehr_seq, emb, W, b):
    idx = ehr_seq.astype(jnp.int32)
    w16 = W.reshape(D).astype(jnp.float32)
    b16 = jnp.broadcast_to(b.astype(jnp.float32), (D,))
    pooled, sig = _ehr_sc(idx, emb, w16, b16)
    embedded = pooled.reshape(1, D)
    output = sig[:1].reshape(1, 1)
    return (output, label, embedded)
